# Initial kernel scaffold; baseline (speedup 1.0000x reference)
import jax
import jax.numpy as jnp
from jax.experimental import pallas as pl


def kernel(depth, points):
    raise NotImplementedError("write your pallas kernel here")


# speedup vs baseline: 43.5982x; 43.5982x over previous
import jax
import jax.numpy as jnp
from jax import lax
from jax.experimental import pallas as pl
from jax.experimental.pallas import tpu as pltpu
from jax.experimental.pallas import tpu_sc as plsc

B, N, H, W = 8, 256, 512, 512
MAX_ITERS = 20
NC, NS, L = 2, 16, 16
NW = NC * NS
PTS = B * N
PPW = PTS // NW
WPB = N // PPW
NBR = 9
OFFS = [(dy, dx) for dy in (-1, 0, 1) for dx in (-1, 0, 1)]


def _smooth_body(d_ref, o_ref):
    a = d_ref[0, 0]
    zr = jnp.zeros((1, W), jnp.float32)
    rs = a + jnp.concatenate([a[1:], zr], 0) + jnp.concatenate([zr, a[:-1]], 0)
    zc = jnp.zeros((H, 1), jnp.float32)
    cs = rs + jnp.concatenate([rs[:, 1:], zc], 1) + jnp.concatenate([zc, rs[:, :-1]], 1)
    o_ref[0] = cs * jnp.float32(1.0 / 9.0)


_smooth_call = pl.pallas_call(
    _smooth_body,
    out_shape=jax.ShapeDtypeStruct((B, H, W), jnp.float32),
    grid=(B,),
    in_specs=[pl.BlockSpec((1, 1, H, W), lambda b: (b, 0, 0, 0))],
    out_specs=pl.BlockSpec((1, H, W), lambda b: (b, 0, 0)),
)


def _climb_body(d_hbm, ys_hbm, xs_hbm, yo_hbm, xo_hbm,
                ycur, xcur, idx_ref, vals_ref, yf, xf, sem):
    wid = lax.axis_index("s") * NC + lax.axis_index("c")
    base_pt = wid * PPW
    boff = (wid // WPB) * (H * W)

    pltpu.sync_copy(ys_hbm.at[pl.ds(base_pt, PPW)], ycur)
    pltpu.sync_copy(xs_hbm.at[pl.ds(base_pt, PPW)], xcur)

    def body(_, carry):
        for g in range(PPW // L):
            yv = ycur[pl.ds(g * L, L)]
            xv = xcur[pl.ds(g * L, L)]
            for k, (dy, dx) in enumerate(OFFS):
                ny = jnp.clip(yv + dy, 0, H - 1)
                nx = jnp.clip(xv + dx, 0, W - 1)
                idx_ref[k, pl.ds(g * L, L)] = boff + ny * W + nx
        copies = [pltpu.async_copy(d_hbm.at[idx_ref.at[k]], vals_ref.at[k], sem)
                  for k in range(NBR)]
        for c in copies:
            c.wait()
        for g in range(PPW // L):
            yv = ycur[pl.ds(g * L, L)]
            xv = xcur[pl.ds(g * L, L)]
            bv = vals_ref[0, pl.ds(g * L, L)]
            bdy = jnp.full((L,), OFFS[0][0], jnp.int32)
            bdx = jnp.full((L,), OFFS[0][1], jnp.int32)
            for k in range(1, NBR):
                dy, dx = OFFS[k]
                v = vals_ref[k, pl.ds(g * L, L)]
                m = v > bv
                bv = jnp.where(m, v, bv)
                bdy = jnp.where(m, jnp.int32(dy), bdy)
                bdx = jnp.where(m, jnp.int32(dx), bdx)
            ycur[pl.ds(g * L, L)] = jnp.clip(yv + bdy, 0, H - 1)
            xcur[pl.ds(g * L, L)] = jnp.clip(xv + bdx, 0, W - 1)
        return carry

    lax.fori_loop(0, MAX_ITERS, body, 0)

    for g in range(PPW // L):
        yf[pl.ds(g * L, L)] = ycur[pl.ds(g * L, L)].astype(jnp.float32)
        xf[pl.ds(g * L, L)] = xcur[pl.ds(g * L, L)].astype(jnp.float32)
    pltpu.sync_copy(yf, yo_hbm.at[pl.ds(base_pt, PPW)])
    pltpu.sync_copy(xf, xo_hbm.at[pl.ds(base_pt, PPW)])


import functools


@functools.cache
def _climb_call():
    return pl.kernel(
        _climb_body,
        out_type=(jax.ShapeDtypeStruct((PTS,), jnp.float32),
                  jax.ShapeDtypeStruct((PTS,), jnp.float32)),
        mesh=plsc.VectorSubcoreMesh(core_axis_name="c", subcore_axis_name="s",
                                    num_cores=NC, num_subcores=NS),
        scratch_types=[
            pltpu.VMEM((PPW,), jnp.int32),
            pltpu.VMEM((PPW,), jnp.int32),
            pltpu.VMEM((NBR, PPW), jnp.int32),
            pltpu.VMEM((NBR, PPW), jnp.float32),
            pltpu.VMEM((PPW,), jnp.float32),
            pltpu.VMEM((PPW,), jnp.float32),
            pltpu.SemaphoreType.DMA,
        ],
    )


def kernel(depth, points):
    d = _smooth_call(depth)
    d_flat = d.reshape(B * H * W)
    pts = points.reshape(PTS, 2).astype(jnp.int32)
    yf, xf = _climb_call()(d_flat, pts[:, 0], pts[:, 1])
    return jnp.stack([yf, xf], axis=-1).reshape(B, N, 2)
